# Initial kernel scaffold; baseline (speedup 1.0000x reference)
import jax
import jax.numpy as jnp
from jax.experimental import pallas as pl


def kernel(l_pos1, l_y1, l_e1, h_pos1, h_e1, enc_W1, enc_b1, enc_W2, enc_b2, conv1_W1, conv1_b1, conv1_W2, conv1_b2, conv1_eps, conv2_W1, conv2_b1, conv2_W2, conv2_b2, conv2_eps):
    raise NotImplementedError("write your pallas kernel here")


# speedup vs baseline: 1.4760x; 1.4760x over previous
import functools

import jax
import jax.numpy as jnp
import numpy as np
from jax import lax
from jax.experimental import pallas as pl
from jax.experimental.pallas import tpu as pltpu
from jax.experimental.pallas import tpu_sc as plsc

_N = 10000
_H = 128
_E = 320000
_NW = 32
_EDGE_CHUNKS = 84
_PER_TILE = _EDGE_CHUNKS * 128
_ACC_ROWS = 10112
_ROWS_PER_TILE = 632
_DUMP = _N + 8
_Q_PAD = 10240
_QB = 256
_G_PAD = 32768

_SIZES = [10080] * 11 + [9840] * 4 + [9760]
_STARTS = [0]
for _s in _SIZES:
    _STARTS.append(_STARTS[-1] + _s)
_WIN_STARTS = _STARTS[:-1] + [160000 + s for s in _STARTS[:-1]]
_WIN_SIZES = _SIZES + _SIZES


def _dot(a, b):
    return jnp.dot(a, b, preferred_element_type=jnp.float32)


@functools.cache
def _sc_mesh():
    return plsc.VectorSubcoreMesh(core_axis_name="c", subcore_axis_name="s")


def _segsum_body(x_hbm, src_hbm, dst_hbm, zero_hbm, out_hbm,
                 src_v, dst_v, rows_v, acc_sh, sem):
    cid = lax.axis_index("c")
    sid = lax.axis_index("s")
    wid = cid * 16 + sid
    r0 = sid * _ROWS_PER_TILE
    pltpu.sync_copy(zero_hbm.at[pl.ds(r0, _ROWS_PER_TILE)],
                    acc_sh.at[pl.ds(r0, _ROWS_PER_TILE)])
    pltpu.sync_copy(src_hbm.at[wid], src_v)
    pltpu.sync_copy(dst_hbm.at[wid], dst_v)
    plsc.subcore_barrier()

    def body(j, carry):
        pltpu.async_copy(x_hbm.at[src_v.at[j]], rows_v, sem).wait()
        pltpu.sync_copy(rows_v, acc_sh.at[dst_v.at[j]], add=True)
        return carry

    lax.fori_loop(0, _EDGE_CHUNKS, body, 0)
    plsc.subcore_barrier()
    pltpu.sync_copy(acc_sh.at[pl.ds(r0, _ROWS_PER_TILE)],
                    out_hbm.at[cid, pl.ds(r0, _ROWS_PER_TILE)])


def _sc_segsum(x, srcp, dstp, zeros):
    k = functools.partial(
        pl.kernel,
        out_type=jax.ShapeDtypeStruct((2, _ACC_ROWS, _H), jnp.float32),
        mesh=_sc_mesh(),
        scratch_types=[
            pltpu.VMEM((_EDGE_CHUNKS, 128), jnp.int32),
            pltpu.VMEM((_EDGE_CHUNKS, 128), jnp.int32),
            pltpu.VMEM((128, _H), jnp.float32),
            pltpu.VMEM_SHARED((_ACC_ROWS, _H), jnp.float32),
            pltpu.SemaphoreType.DMA,
        ],
    )(_segsum_body)
    return k(x, srcp, dstp, zeros)


def _gather_body(x_hbm, idx_hbm, out_hbm, idx_v, rows_v, sem):
    cid = lax.axis_index("c")
    sid = lax.axis_index("s")
    wid = cid * 16 + sid
    pltpu.sync_copy(idx_hbm.at[wid], idx_v)

    def body(j, carry):
        pltpu.async_copy(x_hbm.at[idx_v.at[j]], rows_v, sem).wait()
        pltpu.sync_copy(rows_v, out_hbm.at[pl.ds(wid * 1024 + j * 128, 128)])
        return carry

    lax.fori_loop(0, 8, body, 0)


def _sc_gather(x, idxp):
    k = functools.partial(
        pl.kernel,
        out_type=jax.ShapeDtypeStruct((_G_PAD, _H), jnp.float32),
        mesh=_sc_mesh(),
        scratch_types=[
            pltpu.VMEM((8, 128), jnp.int32),
            pltpu.VMEM((128, _H), jnp.float32),
            pltpu.SemaphoreType.DMA,
        ],
    )(_gather_body)
    return k(x, idxp)


def _enc1_body(y_ref, p_ref, w1_ref, b1_ref, o_ref):
    x = jnp.concatenate([y_ref[...], p_ref[...]], axis=1)
    o_ref[...] = jax.nn.relu(_dot(x, w1_ref[...]) + b1_ref[...])


def _dotrelu_body(a_ref, w_ref, b_ref, o_ref):
    o_ref[...] = jax.nn.relu(_dot(a_ref[...], w_ref[...]) + b_ref[...])


def _dotbias_body(a_ref, w_ref, b_ref, o_ref):
    o_ref[...] = _dot(a_ref[...], w_ref[...]) + b_ref[...]


def _knn_body(py_ref, pxT_ref, idx_ref, md_ref):
    d2 = None
    for d in range(3):
        diff = py_ref[:, d:d + 1] - pxT_ref[d:d + 1, :]
        sq = diff * diff
        d2 = sq if d2 is None else d2 + sq
    lane = lax.broadcasted_iota(jnp.int32, d2.shape, 1)
    inf = jnp.float32(jnp.inf)
    d2 = jnp.where(lane >= _N, inf, d2)
    big = jnp.int32(2 ** 30)
    idxs, ms = [], []
    for _ in range(3):
        m = jnp.min(d2, axis=1, keepdims=True)
        ir = jnp.min(jnp.where(d2 <= m, lane, big), axis=1, keepdims=True)
        ms.append(m)
        idxs.append(ir)
        d2 = jnp.where(lane == ir, inf, d2)
    idx_ref[...] = jnp.concatenate(idxs, axis=1)
    md_ref[...] = jnp.concatenate(ms, axis=1)


def _prep_edges(e):
    order = jnp.argsort(e[1], stable=True)
    src_s = e[0][order]
    dst_s = jnp.asarray(e[1])[order]
    srcw, dstw, headsrc, headmask, firsts = [], [], [], [], []
    for t in range(_NW):
        lo, n = _WIN_STARTS[t], _WIN_SIZES[t]
        ss = lax.slice(src_s, (lo,), (lo + n,))
        ds = lax.slice(dst_s, (lo,), (lo + n,))
        fd = ds[0]
        ishead = ds == fd
        pad = _PER_TILE - n
        srcw.append(jnp.concatenate([ss, jnp.zeros((pad,), jnp.int32)]))
        dstw.append(jnp.concatenate([jnp.where(ishead, _DUMP, ds),
                                     jnp.full((pad,), _DUMP, jnp.int32)]))
        headsrc.append(ss[:128])
        headmask.append(ishead[:128])
        firsts.append(fd)
    return (jnp.stack(srcw).reshape(_NW, _EDGE_CHUNKS, 128),
            jnp.stack(dstw).reshape(_NW, _EDGE_CHUNKS, 128),
            jnp.stack(headsrc), jnp.stack(headmask), jnp.stack(firsts))


def _segsum(x, ep):
    srcp, dstp, headsrc, headmask, firsts = ep
    zeros = jnp.zeros((_ACC_ROWS, _H), jnp.float32)
    acc2 = _sc_segsum(x, srcp, dstp, zeros)
    agg = (acc2[0] + acc2[1])[:_N]
    xh = x[headsrc]
    hp = jnp.zeros((_NW, _H), jnp.float32)
    for j in range(128):
        hp = hp + jnp.where(headmask[:, j:j + 1], xh[:, j, :], 0.0)
    return agg.at[firsts].add(hp)


def _gin_stack(x, ep, W1, b1, W2, b2, eps):
    os_ = jax.ShapeDtypeStruct((_N, _H), jnp.float32)
    for d in range(3):
        agg = _segsum(x, ep)
        agg = agg - jnp.mean(agg, 0)
        out = agg + (1.0 + eps[d]) * x
        h = pl.pallas_call(_dotrelu_body, out_shape=os_)(
            out, W1[d], b1[d].reshape(1, _H))
        y = pl.pallas_call(_dotbias_body, out_shape=os_)(
            h, W2[d], b2[d].reshape(1, _H))
        x = x + jax.nn.relu(y)
        x = x - jnp.mean(x, 0)
    return x


def kernel(l_pos1, l_y1, l_e1, h_pos1, h_e1, enc_W1, enc_b1, enc_W2, enc_b2,
           conv1_W1, conv1_b1, conv1_W2, conv1_b2, conv1_eps,
           conv2_W1, conv2_b1, conv2_W2, conv2_b2, conv2_eps):
    os_ = jax.ShapeDtypeStruct((_N, _H), jnp.float32)
    h0 = pl.pallas_call(_enc1_body, out_shape=os_)(
        l_y1, l_pos1, enc_W1, enc_b1.reshape(1, _H))
    x = pl.pallas_call(_dotbias_body, out_shape=os_)(
        h0, enc_W2, enc_b2.reshape(1, _H))

    epl = _prep_edges(l_e1)
    x = _gin_stack(x, epl, conv1_W1, conv1_b1, conv1_W2, conv1_b2, conv1_eps)

    py = jnp.concatenate([h_pos1, jnp.zeros((_Q_PAD - _N, 3), jnp.float32)], 0)
    pxT = jnp.concatenate([l_pos1.T, jnp.zeros((3, _Q_PAD - _N), jnp.float32)], 1)
    idx3, m3 = pl.pallas_call(
        _knn_body,
        grid=(_Q_PAD // _QB,),
        in_specs=[pl.BlockSpec((_QB, 3), lambda i: (i, 0)),
                  pl.BlockSpec((3, _Q_PAD), lambda i: (0, 0))],
        out_specs=[pl.BlockSpec((_QB, 3), lambda i: (i, 0)),
                   pl.BlockSpec((_QB, 3), lambda i: (i, 0))],
        out_shape=[jax.ShapeDtypeStruct((_Q_PAD, 3), jnp.int32),
                   jax.ShapeDtypeStruct((_Q_PAD, 3), jnp.float32)],
    )(py, pxT)

    flat_idx = jnp.concatenate(
        [idx3.reshape(-1), jnp.zeros((_G_PAD - 3 * _Q_PAD,), jnp.int32)]
    ).reshape(_NW, 8, 128)
    rows = _sc_gather(x, flat_idx)
    xi = rows[:3 * _Q_PAD].reshape(_Q_PAD, 3, _H)[:_N]
    w = 1.0 / jnp.maximum(m3[:_N], 1e-16)
    x2 = jnp.sum(xi * w[..., None], 1) / jnp.sum(w, 1, keepdims=True)

    eph = _prep_edges(h_e1)
    x2 = _gin_stack(x2, eph, conv2_W1, conv2_b1, conv2_W2, conv2_b2, conv2_eps)
    return x2
